# Initial kernel scaffold; baseline (speedup 1.0000x reference)
import jax
import jax.numpy as jnp
from jax.experimental import pallas as pl


def kernel(xyz_f1, points_f1, xyz_f2, points_f2, xyz_f1_prev, points_f1_prev, embedding_mask_prev, q_prev, t_prev, params):
    raise NotImplementedError("write your pallas kernel here")


# speedup vs baseline: 13.8488x; 13.8488x over previous
import functools

import jax
import jax.numpy as jnp
from jax import lax
from jax.experimental import pallas as pl

F32 = jnp.float32
_B, _N1, _N2, _N3 = 4, 2048, 2048, 512
_Q = 512


def _dot(a, b):
    return jnp.dot(a, b, preferred_element_type=F32)


def _relu(x):
    return jnp.maximum(x, 0.0)


def _dist(q, r):
    cross = lax.dot_general(q, r, (((1,), (1,)), ((), ())),
                            preferred_element_type=F32)
    return (jnp.sum(q * q, axis=1, keepdims=True) - 2.0 * cross
            + jnp.sum(r * r, axis=1)[None, :])


def _argmin_onehot(d):
    nr = d.shape[1]
    m = jnp.min(d, axis=1, keepdims=True)
    iota = lax.broadcasted_iota(jnp.int32, d.shape, 1)
    idx = jnp.min(jnp.where(d == m, iota, nr), axis=1, keepdims=True)
    return iota == idx


def _qmul_c(a, b):
    ax, ay, az, aw = a
    bx, by, bz, bw = b
    return (aw * bx + ax * bw + ay * bz - az * by,
            aw * by - ax * bz + ay * bw + az * bx,
            aw * bz + ax * by - ay * bx + az * bw,
            aw * bw - ax * bx - ay * by - az * bz)


def _warp_cols(px, py, pz, q, tx, ty, tz):
    qn = q / (jnp.sqrt(jnp.sum(q * q, axis=1, keepdims=True)) + 1e-10)
    qx, qy, qz, qw = qn[:, 0:1], qn[:, 1:2], qn[:, 2:3], qn[:, 3:4]
    zero = jnp.zeros_like(px)
    r = _qmul_c((qx, qy, qz, qw), (px, py, pz, zero))
    r = _qmul_c(r, (-qx, -qy, -qz, qw))
    return r[0] + tx, r[1] + ty, r[2] + tz


def _wspec(shape):
    return pl.BlockSpec(shape, lambda b, i: tuple(0 for _ in shape))


def _wspec1(shape):
    return pl.BlockSpec(shape, lambda b: tuple(0 for _ in shape))


def _flat(params_list):
    out = []
    for (w, bb) in params_list:
        out.append(w)
        out.append(bb.reshape(1, -1))
    return out


def _suc_kernel(x1_ref, x2_ref, f2_ref, p1_ref,
                w1, b1, w2, b2, wp, bp, o_ref):
    x1 = x1_ref[0]
    x2 = x2_ref[0]
    f2 = f2_ref[0]
    d = _dist(x1, x2)
    hmax = jnp.full((x1.shape[0], w2.shape[1]), -jnp.inf, F32)
    for _ in range(8):
        oh = _argmin_onehot(d)
        ohf = jnp.where(oh, 1.0, 0.0).astype(F32)
        g_xyz = _dot(ohf, x2) - x1
        g_feat = _dot(ohf, f2)
        h = jnp.concatenate([g_feat, g_xyz], axis=1)
        h = _relu(_dot(h, w1[...]) + b1[...])
        h = _relu(_dot(h, w2[...]) + b2[...])
        hmax = jnp.maximum(hmax, h)
        d = jnp.where(oh, jnp.inf, d)
    hp = jnp.concatenate([hmax, p1_ref[0]], axis=1)
    o_ref[0] = _relu(_dot(hp, wp[...]) + bp[...])


def _setupconv(x1t, x2t, f2t, p1t, mlp, post):
    cf = f2t.shape[-1]
    nblk = _N1 // _Q
    ws = _flat(mlp) + _flat(post)
    return pl.pallas_call(
        _suc_kernel,
        grid=(_B, nblk),
        in_specs=[
            pl.BlockSpec((1, _Q, 3), lambda b, i: (b, i, 0)),
            pl.BlockSpec((1, _N3, 3), lambda b, i: (b, 0, 0)),
            pl.BlockSpec((1, _N3, cf), lambda b, i: (b, 0, 0)),
            pl.BlockSpec((1, _Q, 64), lambda b, i: (b, i, 0)),
        ] + [_wspec(w.shape) for w in ws],
        out_specs=pl.BlockSpec((1, _Q, 64), lambda b, i: (b, i, 0)),
        out_shape=jax.ShapeDtypeStruct((_B, _N1, 64), F32),
    )(x1t, x2t, f2t, p1t, *ws)


def _cv1_kernel(x1_ref, x2_ref, f2_ref, p1_ref, q_ref, t_ref,
                wm1, bm1, wm2, bm2, wm3, bm3, we, be,
                wq1, bq1, wq2, bq2, pif_ref, xw_ref):
    x1 = x1_ref[0]
    q = q_ref[0]
    t = t_ref[0]
    wx, wy, wz = _warp_cols(x1[:, 0:1], x1[:, 1:2], x1[:, 2:3], q,
                            t[:, 0:1], t[:, 1:2], t[:, 2:3])
    xw = jnp.concatenate([wx, wy, wz], axis=1)
    x2 = x2_ref[0]
    f2 = f2_ref[0]
    p1 = p1_ref[0]
    d = _dist(xw, x2)
    feats = []
    wqs = []
    for _ in range(6):
        oh = _argmin_onehot(d)
        ohf = jnp.where(oh, 1.0, 0.0).astype(F32)
        qi_xyz = _dot(ohf, x2)
        qi_f = _dot(ohf, f2)
        diff = qi_xyz - xw
        euc = jnp.sqrt(jnp.sum(diff * diff, axis=1, keepdims=True) + 1e-10)
        xyz_cat = jnp.concatenate([xw, qi_xyz, diff, euc], axis=1)
        h = jnp.concatenate([xyz_cat, p1, qi_f], axis=1)
        h = _relu(_dot(h, wm1[...]) + bm1[...])
        h = _relu(_dot(h, wm2[...]) + bm2[...])
        feat = _relu(_dot(h, wm3[...]) + bm3[...])
        enc = _relu(_dot(xyz_cat, we[...]) + be[...])
        hq = jnp.concatenate([enc, feat], axis=1)
        hq = _relu(_dot(hq, wq1[...]) + bq1[...])
        wq = _relu(_dot(hq, wq2[...]) + bq2[...])
        feats.append(feat)
        wqs.append(wq)
        d = jnp.where(oh, jnp.inf, d)
    wmax = functools.reduce(jnp.maximum, wqs)
    es = [jnp.exp(w - wmax) for w in wqs]
    ssum = functools.reduce(jnp.add, es)
    pif = functools.reduce(jnp.add, [e * f for e, f in zip(es, feats)]) / ssum
    pif_ref[0] = pif
    xw_ref[0] = xw


def _cv1(x1t, x2t, f2t, p1t, q3, t3, p):
    nblk = _N1 // _Q
    ws = _flat(p["cv_mlp1"]) + _flat(p["cv_enc1"]) + _flat(p["cv_w_q"])
    return pl.pallas_call(
        _cv1_kernel,
        grid=(_B, nblk),
        in_specs=[
            pl.BlockSpec((1, _Q, 3), lambda b, i: (b, i, 0)),
            pl.BlockSpec((1, _N2, 3), lambda b, i: (b, 0, 0)),
            pl.BlockSpec((1, _N2, 64), lambda b, i: (b, 0, 0)),
            pl.BlockSpec((1, _Q, 64), lambda b, i: (b, i, 0)),
            pl.BlockSpec((1, 1, 4), lambda b, i: (b, 0, 0)),
            pl.BlockSpec((1, 1, 3), lambda b, i: (b, 0, 0)),
        ] + [_wspec(w.shape) for w in ws],
        out_specs=[
            pl.BlockSpec((1, _Q, 64), lambda b, i: (b, i, 0)),
            pl.BlockSpec((1, _Q, 3), lambda b, i: (b, i, 0)),
        ],
        out_shape=[
            jax.ShapeDtypeStruct((_B, _N1, 64), F32),
            jax.ShapeDtypeStruct((_B, _N1, 3), F32),
        ],
    )(x1t, x2t, f2t, p1t, q3, t3, *ws)


def _cv2_kernel(xw_ref, xr_ref, fr_ref, p1_ref,
                we, be, wp1, bp1, wp2, bp2, o_ref):
    xw = xw_ref[0]
    xr = xr_ref[0]
    fr = fr_ref[0]
    p1 = p1_ref[0]
    d = _dist(xw, xr)
    gs = []
    wps = []
    for _ in range(4):
        oh = _argmin_onehot(d)
        ohf = jnp.where(oh, 1.0, 0.0).astype(F32)
        pc_xyz = _dot(ohf, xr)
        pc_g = _dot(ohf, fr)
        d2 = pc_xyz - xw
        e2 = jnp.sqrt(jnp.sum(d2 * d2, axis=1, keepdims=True) + 1e-10)
        xyz_cat2 = jnp.concatenate([xw, pc_xyz, d2, e2], axis=1)
        enc2 = _relu(_dot(xyz_cat2, we[...]) + be[...])
        h = jnp.concatenate([enc2, p1, pc_g], axis=1)
        h = _relu(_dot(h, wp1[...]) + bp1[...])
        wpv = _relu(_dot(h, wp2[...]) + bp2[...])
        gs.append(pc_g)
        wps.append(wpv)
        d = jnp.where(oh, jnp.inf, d)
    wmax = functools.reduce(jnp.maximum, wps)
    es = [jnp.exp(w - wmax) for w in wps]
    ssum = functools.reduce(jnp.add, es)
    o_ref[0] = functools.reduce(
        jnp.add, [e * g for e, g in zip(es, gs)]) / ssum


def _cv2(xw, pif, p1t, p):
    nblk = _N1 // _Q
    ws = _flat(p["cv_enc2"]) + _flat(p["cv_w_p"])
    return pl.pallas_call(
        _cv2_kernel,
        grid=(_B, nblk),
        in_specs=[
            pl.BlockSpec((1, _Q, 3), lambda b, i: (b, i, 0)),
            pl.BlockSpec((1, _N1, 3), lambda b, i: (b, 0, 0)),
            pl.BlockSpec((1, _N1, 64), lambda b, i: (b, 0, 0)),
            pl.BlockSpec((1, _Q, 64), lambda b, i: (b, i, 0)),
        ] + [_wspec(w.shape) for w in ws],
        out_specs=pl.BlockSpec((1, _Q, 64), lambda b, i: (b, i, 0)),
        out_shape=jax.ShapeDtypeStruct((_B, _N1, 64), F32),
    )(xw, xw, pif, p1t, *ws)


def _head_kernel(p1_ref, res_ref, cf_ref, cm_ref, q_ref, t_ref,
                 wf1, bf1, wf2, bf2, wm1, bm1, wm2, bm2,
                 wfc, bfc, whq, bhq, wht, bht,
                 ef_ref, em_ref, qo_ref, to_ref):
    p1 = p1_ref[0]
    res = res_ref[0]
    cfe = cf_ref[0]
    cma = cm_ref[0]
    h = jnp.concatenate([p1, res, cfe], axis=1)
    h = _relu(_dot(h, wf1[...]) + bf1[...])
    ef = _relu(_dot(h, wf2[...]) + bf2[...])
    h = jnp.concatenate([cma, ef, p1], axis=1)
    h = _relu(_dot(h, wm1[...]) + bm1[...])
    em = _relu(_dot(h, wm2[...]) + bm2[...])
    ef_ref[0] = ef
    em_ref[0] = em
    mx = jnp.max(em, axis=0, keepdims=True)
    e = jnp.exp(em - mx)
    wcv = e / jnp.sum(e, axis=0, keepdims=True)
    hsum = jnp.sum(ef * wcv, axis=0, keepdims=True)
    hp = _dot(hsum, wfc[...]) + bfc[...]
    qd = _dot(hp, whq[...]) + bhq[...]
    td = _dot(hp, wht[...]) + bht[...]
    qd = qd / (jnp.sqrt(jnp.sum(qd * qd, axis=1, keepdims=True)) + 1e-10)
    qc = q_ref[0]
    a = (qd[:, 0:1], qd[:, 1:2], qd[:, 2:3], qd[:, 3:4])
    b = (qc[:, 0:1], qc[:, 1:2], qc[:, 2:3], qc[:, 3:4])
    qx, qy, qz, qw = _qmul_c(a, b)
    qo_ref[0] = jnp.concatenate([qx, qy, qz, qw], axis=1)
    tc = t_ref[0]
    tx, ty, tz = _warp_cols(tc[:, 0:1], tc[:, 1:2], tc[:, 2:3], qd,
                            td[:, 0:1], td[:, 1:2], td[:, 2:3])
    to_ref[0] = jnp.concatenate([tx, ty, tz], axis=1)


def _head(p1t, res, cfe, cma, q3, t3, p):
    ws = (_flat(p["fp_feat"]) + _flat(p["fp_mask"]) + _flat(p["pose_fc"])
          + _flat(p["head_q"]) + _flat(p["head_t"]))
    return pl.pallas_call(
        _head_kernel,
        grid=(_B,),
        in_specs=[
            pl.BlockSpec((1, _N1, 64), lambda b: (b, 0, 0)),
            pl.BlockSpec((1, _N1, 64), lambda b: (b, 0, 0)),
            pl.BlockSpec((1, _N1, 64), lambda b: (b, 0, 0)),
            pl.BlockSpec((1, _N1, 64), lambda b: (b, 0, 0)),
            pl.BlockSpec((1, 1, 4), lambda b: (b, 0, 0)),
            pl.BlockSpec((1, 1, 3), lambda b: (b, 0, 0)),
        ] + [_wspec1(w.shape) for w in ws],
        out_specs=[
            pl.BlockSpec((1, _N1, 64), lambda b: (b, 0, 0)),
            pl.BlockSpec((1, _N1, 64), lambda b: (b, 0, 0)),
            pl.BlockSpec((1, 1, 4), lambda b: (b, 0, 0)),
            pl.BlockSpec((1, 1, 3), lambda b: (b, 0, 0)),
        ],
        out_shape=[
            jax.ShapeDtypeStruct((_B, _N1, 64), F32),
            jax.ShapeDtypeStruct((_B, _N1, 64), F32),
            jax.ShapeDtypeStruct((_B, 1, 4), F32),
            jax.ShapeDtypeStruct((_B, 1, 3), F32),
        ],
    )(p1t, res, cfe, cma, q3, t3, *ws)


def kernel(xyz_f1, points_f1, xyz_f2, points_f2, xyz_f1_prev,
           points_f1_prev, embedding_mask_prev, q_prev, t_prev, params):
    x1t = xyz_f1.transpose(0, 2, 1)
    xpt = xyz_f1_prev.transpose(0, 2, 1)
    p1t = points_f1.transpose(0, 2, 1)
    coarse_feat = _setupconv(x1t, xpt, points_f1_prev.transpose(0, 2, 1),
                             p1t, params["suc_feat_mlp"],
                             params["suc_feat_post"])
    coarse_mask = _setupconv(x1t, xpt,
                             embedding_mask_prev.transpose(0, 2, 1),
                             p1t, params["suc_mask_mlp"],
                             params["suc_mask_post"])
    q3 = q_prev.reshape(_B, 1, 4)
    t3 = t_prev.reshape(_B, 1, 3)
    pi_feat, xw = _cv1(x1t, xyz_f2.transpose(0, 2, 1),
                       points_f2.transpose(0, 2, 1), p1t, q3, t3, params)
    residual = _cv2(xw, pi_feat, p1t, params)
    ef, em, qo, to = _head(p1t, residual, coarse_feat, coarse_mask,
                           q3, t3, params)
    return (qo.reshape(_B, 4), to.reshape(_B, 3),
            ef.transpose(0, 2, 1), em.transpose(0, 2, 1))


# speedup vs baseline: 14.1114x; 1.0190x over previous
import functools

import jax
import jax.numpy as jnp
from jax import lax
from jax.experimental import pallas as pl

F32 = jnp.float32
_B, _N1, _N2, _N3 = 4, 2048, 2048, 512
_Q = 512


def _dot(a, b):
    return jnp.dot(a, b, preferred_element_type=F32)


def _relu(x):
    return jnp.maximum(x, 0.0)


def _dist(q, r):
    cross = lax.dot_general(q, r, (((1,), (1,)), ((), ())),
                            preferred_element_type=F32)
    return (jnp.sum(q * q, axis=1, keepdims=True) - 2.0 * cross
            + jnp.sum(r * r, axis=1)[None, :])


def _argmin_onehot(d, iota):
    nr = d.shape[1]
    m = jnp.min(d, axis=1, keepdims=True)
    idx = jnp.min(jnp.where(d == m, iota, nr), axis=1, keepdims=True)
    return iota == idx


def _qmul_c(a, b):
    ax, ay, az, aw = a
    bx, by, bz, bw = b
    return (aw * bx + ax * bw + ay * bz - az * by,
            aw * by - ax * bz + ay * bw + az * bx,
            aw * bz + ax * by - ay * bx + az * bw,
            aw * bw - ax * bx - ay * by - az * bz)


def _warp_cols(px, py, pz, q, tx, ty, tz):
    qn = q / (jnp.sqrt(jnp.sum(q * q, axis=1, keepdims=True)) + 1e-10)
    qx, qy, qz, qw = qn[:, 0:1], qn[:, 1:2], qn[:, 2:3], qn[:, 3:4]
    zero = jnp.zeros_like(px)
    r = _qmul_c((qx, qy, qz, qw), (px, py, pz, zero))
    r = _qmul_c(r, (-qx, -qy, -qz, qw))
    return r[0] + tx, r[1] + ty, r[2] + tz


def _wspec(shape):
    return pl.BlockSpec(shape, lambda b, i: tuple(0 for _ in shape))


def _wspec1(shape):
    return pl.BlockSpec(shape, lambda b: tuple(0 for _ in shape))


def _flat(params_list):
    out = []
    for (w, bb) in params_list:
        out.append(w)
        out.append(bb.reshape(1, -1))
    return out


def _suc_kernel(x1_ref, x2_ref, f2_ref, p1_ref,
                w1, b1, w2, b2, wp, bp, o_ref):
    x1 = x1_ref[0]
    x2 = x2_ref[0]
    f2 = f2_ref[0]
    cf = f2.shape[1]
    fx2 = jnp.concatenate([f2, x2], axis=1)
    x1pad = jnp.concatenate(
        [jnp.zeros((x1.shape[0], cf), F32), x1], axis=1)
    d = _dist(x1, x2)
    iota = lax.broadcasted_iota(jnp.int32, d.shape, 1)
    hmax = jnp.full((x1.shape[0], w2.shape[1]), -jnp.inf, F32)
    for _ in range(8):
        oh = _argmin_onehot(d, iota)
        ohf = jnp.where(oh, 1.0, 0.0).astype(F32)
        h = _dot(ohf, fx2) - x1pad
        h = _relu(_dot(h, w1[...]) + b1[...])
        h = _relu(_dot(h, w2[...]) + b2[...])
        hmax = jnp.maximum(hmax, h)
        d = jnp.where(oh, jnp.inf, d)
    hp = jnp.concatenate([hmax, p1_ref[0]], axis=1)
    o_ref[0] = _relu(_dot(hp, wp[...]) + bp[...])


def _setupconv(x1t, x2t, f2t, p1t, mlp, post):
    cf = f2t.shape[-1]
    nblk = _N1 // _Q
    ws = _flat(mlp) + _flat(post)
    return pl.pallas_call(
        _suc_kernel,
        grid=(_B, nblk),
        in_specs=[
            pl.BlockSpec((1, _Q, 3), lambda b, i: (b, i, 0)),
            pl.BlockSpec((1, _N3, 3), lambda b, i: (b, 0, 0)),
            pl.BlockSpec((1, _N3, cf), lambda b, i: (b, 0, 0)),
            pl.BlockSpec((1, _Q, 64), lambda b, i: (b, i, 0)),
        ] + [_wspec(w.shape) for w in ws],
        out_specs=pl.BlockSpec((1, _Q, 64), lambda b, i: (b, i, 0)),
        out_shape=jax.ShapeDtypeStruct((_B, _N1, 64), F32),
    )(x1t, x2t, f2t, p1t, *ws)


def _cv1_kernel(x1_ref, x2_ref, f2_ref, p1_ref, q_ref, t_ref,
                wm1, bm1, wm2, bm2, wm3, bm3, we, be,
                wq1, bq1, wq2, bq2, pif_ref, xw_ref):
    x1 = x1_ref[0]
    q = q_ref[0]
    t = t_ref[0]
    wx, wy, wz = _warp_cols(x1[:, 0:1], x1[:, 1:2], x1[:, 2:3], q,
                            t[:, 0:1], t[:, 1:2], t[:, 2:3])
    xw = jnp.concatenate([wx, wy, wz], axis=1)
    x2 = x2_ref[0]
    f2 = f2_ref[0]
    p1 = p1_ref[0]
    xf2 = jnp.concatenate([x2, f2], axis=1)
    d = _dist(xw, x2)
    iota = lax.broadcasted_iota(jnp.int32, d.shape, 1)
    feats = []
    wqs = []
    for _ in range(6):
        oh = _argmin_onehot(d, iota)
        ohf = jnp.where(oh, 1.0, 0.0).astype(F32)
        g = _dot(ohf, xf2)
        qi_xyz = g[:, 0:3]
        qi_f = g[:, 3:]
        diff = qi_xyz - xw
        euc = jnp.sqrt(jnp.sum(diff * diff, axis=1, keepdims=True) + 1e-10)
        xyz_cat = jnp.concatenate([xw, qi_xyz, diff, euc], axis=1)
        h = jnp.concatenate([xyz_cat, p1, qi_f], axis=1)
        h = _relu(_dot(h, wm1[...]) + bm1[...])
        h = _relu(_dot(h, wm2[...]) + bm2[...])
        feat = _relu(_dot(h, wm3[...]) + bm3[...])
        enc = _relu(_dot(xyz_cat, we[...]) + be[...])
        hq = jnp.concatenate([enc, feat], axis=1)
        hq = _relu(_dot(hq, wq1[...]) + bq1[...])
        wq = _relu(_dot(hq, wq2[...]) + bq2[...])
        feats.append(feat)
        wqs.append(wq)
        d = jnp.where(oh, jnp.inf, d)
    wmax = functools.reduce(jnp.maximum, wqs)
    es = [jnp.exp(w - wmax) for w in wqs]
    ssum = functools.reduce(jnp.add, es)
    pif = functools.reduce(jnp.add, [e * f for e, f in zip(es, feats)]) / ssum
    pif_ref[0] = pif
    xw_ref[0] = xw


def _cv1(x1t, x2t, f2t, p1t, q3, t3, p):
    nblk = _N1 // _Q
    ws = _flat(p["cv_mlp1"]) + _flat(p["cv_enc1"]) + _flat(p["cv_w_q"])
    return pl.pallas_call(
        _cv1_kernel,
        grid=(_B, nblk),
        in_specs=[
            pl.BlockSpec((1, _Q, 3), lambda b, i: (b, i, 0)),
            pl.BlockSpec((1, _N2, 3), lambda b, i: (b, 0, 0)),
            pl.BlockSpec((1, _N2, 64), lambda b, i: (b, 0, 0)),
            pl.BlockSpec((1, _Q, 64), lambda b, i: (b, i, 0)),
            pl.BlockSpec((1, 1, 4), lambda b, i: (b, 0, 0)),
            pl.BlockSpec((1, 1, 3), lambda b, i: (b, 0, 0)),
        ] + [_wspec(w.shape) for w in ws],
        out_specs=[
            pl.BlockSpec((1, _Q, 64), lambda b, i: (b, i, 0)),
            pl.BlockSpec((1, _Q, 3), lambda b, i: (b, i, 0)),
        ],
        out_shape=[
            jax.ShapeDtypeStruct((_B, _N1, 64), F32),
            jax.ShapeDtypeStruct((_B, _N1, 3), F32),
        ],
    )(x1t, x2t, f2t, p1t, q3, t3, *ws)


def _cv2_kernel(xw_ref, xr_ref, fr_ref, p1_ref,
                we, be, wp1, bp1, wp2, bp2, o_ref):
    xw = xw_ref[0]
    xr = xr_ref[0]
    fr = fr_ref[0]
    p1 = p1_ref[0]
    xfr = jnp.concatenate([xr, fr], axis=1)
    d = _dist(xw, xr)
    iota = lax.broadcasted_iota(jnp.int32, d.shape, 1)
    gs = []
    wps = []
    for _ in range(4):
        oh = _argmin_onehot(d, iota)
        ohf = jnp.where(oh, 1.0, 0.0).astype(F32)
        g = _dot(ohf, xfr)
        pc_xyz = g[:, 0:3]
        pc_g = g[:, 3:]
        d2 = pc_xyz - xw
        e2 = jnp.sqrt(jnp.sum(d2 * d2, axis=1, keepdims=True) + 1e-10)
        xyz_cat2 = jnp.concatenate([xw, pc_xyz, d2, e2], axis=1)
        enc2 = _relu(_dot(xyz_cat2, we[...]) + be[...])
        h = jnp.concatenate([enc2, p1, pc_g], axis=1)
        h = _relu(_dot(h, wp1[...]) + bp1[...])
        wpv = _relu(_dot(h, wp2[...]) + bp2[...])
        gs.append(pc_g)
        wps.append(wpv)
        d = jnp.where(oh, jnp.inf, d)
    wmax = functools.reduce(jnp.maximum, wps)
    es = [jnp.exp(w - wmax) for w in wps]
    ssum = functools.reduce(jnp.add, es)
    o_ref[0] = functools.reduce(
        jnp.add, [e * g for e, g in zip(es, gs)]) / ssum


def _cv2(xw, pif, p1t, p):
    nblk = _N1 // _Q
    ws = _flat(p["cv_enc2"]) + _flat(p["cv_w_p"])
    return pl.pallas_call(
        _cv2_kernel,
        grid=(_B, nblk),
        in_specs=[
            pl.BlockSpec((1, _Q, 3), lambda b, i: (b, i, 0)),
            pl.BlockSpec((1, _N1, 3), lambda b, i: (b, 0, 0)),
            pl.BlockSpec((1, _N1, 64), lambda b, i: (b, 0, 0)),
            pl.BlockSpec((1, _Q, 64), lambda b, i: (b, i, 0)),
        ] + [_wspec(w.shape) for w in ws],
        out_specs=pl.BlockSpec((1, _Q, 64), lambda b, i: (b, i, 0)),
        out_shape=jax.ShapeDtypeStruct((_B, _N1, 64), F32),
    )(xw, xw, pif, p1t, *ws)


def _head_kernel(p1_ref, res_ref, cf_ref, cm_ref, q_ref, t_ref,
                 wf1, bf1, wf2, bf2, wm1, bm1, wm2, bm2,
                 wfc, bfc, whq, bhq, wht, bht,
                 ef_ref, em_ref, qo_ref, to_ref):
    p1 = p1_ref[0]
    res = res_ref[0]
    cfe = cf_ref[0]
    cma = cm_ref[0]
    h = jnp.concatenate([p1, res, cfe], axis=1)
    h = _relu(_dot(h, wf1[...]) + bf1[...])
    ef = _relu(_dot(h, wf2[...]) + bf2[...])
    h = jnp.concatenate([cma, ef, p1], axis=1)
    h = _relu(_dot(h, wm1[...]) + bm1[...])
    em = _relu(_dot(h, wm2[...]) + bm2[...])
    ef_ref[0] = ef
    em_ref[0] = em
    mx = jnp.max(em, axis=0, keepdims=True)
    e = jnp.exp(em - mx)
    wcv = e / jnp.sum(e, axis=0, keepdims=True)
    hsum = jnp.sum(ef * wcv, axis=0, keepdims=True)
    hp = _dot(hsum, wfc[...]) + bfc[...]
    qd = _dot(hp, whq[...]) + bhq[...]
    td = _dot(hp, wht[...]) + bht[...]
    qd = qd / (jnp.sqrt(jnp.sum(qd * qd, axis=1, keepdims=True)) + 1e-10)
    qc = q_ref[0]
    a = (qd[:, 0:1], qd[:, 1:2], qd[:, 2:3], qd[:, 3:4])
    b = (qc[:, 0:1], qc[:, 1:2], qc[:, 2:3], qc[:, 3:4])
    qx, qy, qz, qw = _qmul_c(a, b)
    qo_ref[0] = jnp.concatenate([qx, qy, qz, qw], axis=1)
    tc = t_ref[0]
    tx, ty, tz = _warp_cols(tc[:, 0:1], tc[:, 1:2], tc[:, 2:3], qd,
                            td[:, 0:1], td[:, 1:2], td[:, 2:3])
    to_ref[0] = jnp.concatenate([tx, ty, tz], axis=1)


def _head(p1t, res, cfe, cma, q3, t3, p):
    ws = (_flat(p["fp_feat"]) + _flat(p["fp_mask"]) + _flat(p["pose_fc"])
          + _flat(p["head_q"]) + _flat(p["head_t"]))
    return pl.pallas_call(
        _head_kernel,
        grid=(_B,),
        in_specs=[
            pl.BlockSpec((1, _N1, 64), lambda b: (b, 0, 0)),
            pl.BlockSpec((1, _N1, 64), lambda b: (b, 0, 0)),
            pl.BlockSpec((1, _N1, 64), lambda b: (b, 0, 0)),
            pl.BlockSpec((1, _N1, 64), lambda b: (b, 0, 0)),
            pl.BlockSpec((1, 1, 4), lambda b: (b, 0, 0)),
            pl.BlockSpec((1, 1, 3), lambda b: (b, 0, 0)),
        ] + [_wspec1(w.shape) for w in ws],
        out_specs=[
            pl.BlockSpec((1, _N1, 64), lambda b: (b, 0, 0)),
            pl.BlockSpec((1, _N1, 64), lambda b: (b, 0, 0)),
            pl.BlockSpec((1, 1, 4), lambda b: (b, 0, 0)),
            pl.BlockSpec((1, 1, 3), lambda b: (b, 0, 0)),
        ],
        out_shape=[
            jax.ShapeDtypeStruct((_B, _N1, 64), F32),
            jax.ShapeDtypeStruct((_B, _N1, 64), F32),
            jax.ShapeDtypeStruct((_B, 1, 4), F32),
            jax.ShapeDtypeStruct((_B, 1, 3), F32),
        ],
    )(p1t, res, cfe, cma, q3, t3, *ws)


def kernel(xyz_f1, points_f1, xyz_f2, points_f2, xyz_f1_prev,
           points_f1_prev, embedding_mask_prev, q_prev, t_prev, params):
    x1t = xyz_f1.transpose(0, 2, 1)
    xpt = xyz_f1_prev.transpose(0, 2, 1)
    p1t = points_f1.transpose(0, 2, 1)
    coarse_feat = _setupconv(x1t, xpt, points_f1_prev.transpose(0, 2, 1),
                             p1t, params["suc_feat_mlp"],
                             params["suc_feat_post"])
    coarse_mask = _setupconv(x1t, xpt,
                             embedding_mask_prev.transpose(0, 2, 1),
                             p1t, params["suc_mask_mlp"],
                             params["suc_mask_post"])
    q3 = q_prev.reshape(_B, 1, 4)
    t3 = t_prev.reshape(_B, 1, 3)
    pi_feat, xw = _cv1(x1t, xyz_f2.transpose(0, 2, 1),
                       points_f2.transpose(0, 2, 1), p1t, q3, t3, params)
    residual = _cv2(xw, pi_feat, p1t, params)
    ef, em, qo, to = _head(p1t, residual, coarse_feat, coarse_mask,
                           q3, t3, params)
    return (qo.reshape(_B, 4), to.reshape(_B, 3),
            ef.transpose(0, 2, 1), em.transpose(0, 2, 1))


# speedup vs baseline: 14.1199x; 1.0006x over previous
import functools

import jax
import jax.numpy as jnp
from jax import lax
from jax.experimental import pallas as pl
from jax.experimental.pallas import tpu as pltpu

_PAR2 = pltpu.CompilerParams(dimension_semantics=("parallel", "parallel"))
_PAR1 = pltpu.CompilerParams(dimension_semantics=("parallel",))

F32 = jnp.float32
_B, _N1, _N2, _N3 = 4, 2048, 2048, 512
_Q = 512


def _dot(a, b):
    return jnp.dot(a, b, preferred_element_type=F32)


def _relu(x):
    return jnp.maximum(x, 0.0)


def _dist(q, r):
    cross = lax.dot_general(q, r, (((1,), (1,)), ((), ())),
                            preferred_element_type=F32)
    return (jnp.sum(q * q, axis=1, keepdims=True) - 2.0 * cross
            + jnp.sum(r * r, axis=1)[None, :])


def _argmin_onehot(d, iota):
    nr = d.shape[1]
    m = jnp.min(d, axis=1, keepdims=True)
    idx = jnp.min(jnp.where(d == m, iota, nr), axis=1, keepdims=True)
    return iota == idx


def _qmul_c(a, b):
    ax, ay, az, aw = a
    bx, by, bz, bw = b
    return (aw * bx + ax * bw + ay * bz - az * by,
            aw * by - ax * bz + ay * bw + az * bx,
            aw * bz + ax * by - ay * bx + az * bw,
            aw * bw - ax * bx - ay * by - az * bz)


def _warp_cols(px, py, pz, q, tx, ty, tz):
    qn = q / (jnp.sqrt(jnp.sum(q * q, axis=1, keepdims=True)) + 1e-10)
    qx, qy, qz, qw = qn[:, 0:1], qn[:, 1:2], qn[:, 2:3], qn[:, 3:4]
    zero = jnp.zeros_like(px)
    r = _qmul_c((qx, qy, qz, qw), (px, py, pz, zero))
    r = _qmul_c(r, (-qx, -qy, -qz, qw))
    return r[0] + tx, r[1] + ty, r[2] + tz


def _wspec(shape):
    return pl.BlockSpec(shape, lambda b, i: tuple(0 for _ in shape))


def _wspec1(shape):
    return pl.BlockSpec(shape, lambda b: tuple(0 for _ in shape))


def _flat(params_list):
    out = []
    for (w, bb) in params_list:
        out.append(w)
        out.append(bb.reshape(1, -1))
    return out


def _suc_kernel(x1_ref, x2_ref, f2_ref, p1_ref,
                w1, b1, w2, b2, wp, bp, o_ref):
    x1 = x1_ref[0]
    x2 = x2_ref[0]
    f2 = f2_ref[0]
    cf = f2.shape[1]
    fx2 = jnp.concatenate([f2, x2], axis=1)
    x1pad = jnp.concatenate(
        [jnp.zeros((x1.shape[0], cf), F32), x1], axis=1)
    d = _dist(x1, x2)
    iota = lax.broadcasted_iota(jnp.int32, d.shape, 1)
    hmax = jnp.full((x1.shape[0], w2.shape[1]), -jnp.inf, F32)
    for _ in range(8):
        oh = _argmin_onehot(d, iota)
        ohf = jnp.where(oh, 1.0, 0.0).astype(F32)
        h = _dot(ohf, fx2) - x1pad
        h = _relu(_dot(h, w1[...]) + b1[...])
        h = _relu(_dot(h, w2[...]) + b2[...])
        hmax = jnp.maximum(hmax, h)
        d = jnp.where(oh, jnp.inf, d)
    hp = jnp.concatenate([hmax, p1_ref[0]], axis=1)
    o_ref[0] = _relu(_dot(hp, wp[...]) + bp[...])


def _setupconv(x1t, x2t, f2t, p1t, mlp, post):
    cf = f2t.shape[-1]
    nblk = _N1 // _Q
    ws = _flat(mlp) + _flat(post)
    return pl.pallas_call(
        _suc_kernel,
        grid=(_B, nblk),
        in_specs=[
            pl.BlockSpec((1, _Q, 3), lambda b, i: (b, i, 0)),
            pl.BlockSpec((1, _N3, 3), lambda b, i: (b, 0, 0)),
            pl.BlockSpec((1, _N3, cf), lambda b, i: (b, 0, 0)),
            pl.BlockSpec((1, _Q, 64), lambda b, i: (b, i, 0)),
        ] + [_wspec(w.shape) for w in ws],
        out_specs=pl.BlockSpec((1, _Q, 64), lambda b, i: (b, i, 0)),
        out_shape=jax.ShapeDtypeStruct((_B, _N1, 64), F32),
        compiler_params=_PAR2,
    )(x1t, x2t, f2t, p1t, *ws)


def _cv1_kernel(x1_ref, x2_ref, f2_ref, p1_ref, q_ref, t_ref,
                wm1, bm1, wm2, bm2, wm3, bm3, we, be,
                wq1, bq1, wq2, bq2, pif_ref, xw_ref):
    x1 = x1_ref[0]
    q = q_ref[0]
    t = t_ref[0]
    wx, wy, wz = _warp_cols(x1[:, 0:1], x1[:, 1:2], x1[:, 2:3], q,
                            t[:, 0:1], t[:, 1:2], t[:, 2:3])
    xw = jnp.concatenate([wx, wy, wz], axis=1)
    x2 = x2_ref[0]
    f2 = f2_ref[0]
    p1 = p1_ref[0]
    xf2 = jnp.concatenate([x2, f2], axis=1)
    d = _dist(xw, x2)
    iota = lax.broadcasted_iota(jnp.int32, d.shape, 1)
    feats = []
    wqs = []
    for _ in range(6):
        oh = _argmin_onehot(d, iota)
        ohf = jnp.where(oh, 1.0, 0.0).astype(F32)
        g = _dot(ohf, xf2)
        qi_xyz = g[:, 0:3]
        qi_f = g[:, 3:]
        diff = qi_xyz - xw
        euc = jnp.sqrt(jnp.sum(diff * diff, axis=1, keepdims=True) + 1e-10)
        xyz_cat = jnp.concatenate([xw, qi_xyz, diff, euc], axis=1)
        h = jnp.concatenate([xyz_cat, p1, qi_f], axis=1)
        h = _relu(_dot(h, wm1[...]) + bm1[...])
        h = _relu(_dot(h, wm2[...]) + bm2[...])
        feat = _relu(_dot(h, wm3[...]) + bm3[...])
        enc = _relu(_dot(xyz_cat, we[...]) + be[...])
        hq = jnp.concatenate([enc, feat], axis=1)
        hq = _relu(_dot(hq, wq1[...]) + bq1[...])
        wq = _relu(_dot(hq, wq2[...]) + bq2[...])
        feats.append(feat)
        wqs.append(wq)
        d = jnp.where(oh, jnp.inf, d)
    wmax = functools.reduce(jnp.maximum, wqs)
    es = [jnp.exp(w - wmax) for w in wqs]
    ssum = functools.reduce(jnp.add, es)
    pif = functools.reduce(jnp.add, [e * f for e, f in zip(es, feats)]) / ssum
    pif_ref[0] = pif
    xw_ref[0] = xw


def _cv1(x1t, x2t, f2t, p1t, q3, t3, p):
    nblk = _N1 // _Q
    ws = _flat(p["cv_mlp1"]) + _flat(p["cv_enc1"]) + _flat(p["cv_w_q"])
    return pl.pallas_call(
        _cv1_kernel,
        grid=(_B, nblk),
        in_specs=[
            pl.BlockSpec((1, _Q, 3), lambda b, i: (b, i, 0)),
            pl.BlockSpec((1, _N2, 3), lambda b, i: (b, 0, 0)),
            pl.BlockSpec((1, _N2, 64), lambda b, i: (b, 0, 0)),
            pl.BlockSpec((1, _Q, 64), lambda b, i: (b, i, 0)),
            pl.BlockSpec((1, 1, 4), lambda b, i: (b, 0, 0)),
            pl.BlockSpec((1, 1, 3), lambda b, i: (b, 0, 0)),
        ] + [_wspec(w.shape) for w in ws],
        out_specs=[
            pl.BlockSpec((1, _Q, 64), lambda b, i: (b, i, 0)),
            pl.BlockSpec((1, _Q, 3), lambda b, i: (b, i, 0)),
        ],
        out_shape=[
            jax.ShapeDtypeStruct((_B, _N1, 64), F32),
            jax.ShapeDtypeStruct((_B, _N1, 3), F32),
        ],
        compiler_params=_PAR2,
    )(x1t, x2t, f2t, p1t, q3, t3, *ws)


def _cv2_kernel(xw_ref, xr_ref, fr_ref, p1_ref,
                we, be, wp1, bp1, wp2, bp2, o_ref):
    xw = xw_ref[0]
    xr = xr_ref[0]
    fr = fr_ref[0]
    p1 = p1_ref[0]
    xfr = jnp.concatenate([xr, fr], axis=1)
    d = _dist(xw, xr)
    iota = lax.broadcasted_iota(jnp.int32, d.shape, 1)
    gs = []
    wps = []
    for _ in range(4):
        oh = _argmin_onehot(d, iota)
        ohf = jnp.where(oh, 1.0, 0.0).astype(F32)
        g = _dot(ohf, xfr)
        pc_xyz = g[:, 0:3]
        pc_g = g[:, 3:]
        d2 = pc_xyz - xw
        e2 = jnp.sqrt(jnp.sum(d2 * d2, axis=1, keepdims=True) + 1e-10)
        xyz_cat2 = jnp.concatenate([xw, pc_xyz, d2, e2], axis=1)
        enc2 = _relu(_dot(xyz_cat2, we[...]) + be[...])
        h = jnp.concatenate([enc2, p1, pc_g], axis=1)
        h = _relu(_dot(h, wp1[...]) + bp1[...])
        wpv = _relu(_dot(h, wp2[...]) + bp2[...])
        gs.append(pc_g)
        wps.append(wpv)
        d = jnp.where(oh, jnp.inf, d)
    wmax = functools.reduce(jnp.maximum, wps)
    es = [jnp.exp(w - wmax) for w in wps]
    ssum = functools.reduce(jnp.add, es)
    o_ref[0] = functools.reduce(
        jnp.add, [e * g for e, g in zip(es, gs)]) / ssum


def _cv2(xw, pif, p1t, p):
    nblk = _N1 // _Q
    ws = _flat(p["cv_enc2"]) + _flat(p["cv_w_p"])
    return pl.pallas_call(
        _cv2_kernel,
        grid=(_B, nblk),
        in_specs=[
            pl.BlockSpec((1, _Q, 3), lambda b, i: (b, i, 0)),
            pl.BlockSpec((1, _N1, 3), lambda b, i: (b, 0, 0)),
            pl.BlockSpec((1, _N1, 64), lambda b, i: (b, 0, 0)),
            pl.BlockSpec((1, _Q, 64), lambda b, i: (b, i, 0)),
        ] + [_wspec(w.shape) for w in ws],
        out_specs=pl.BlockSpec((1, _Q, 64), lambda b, i: (b, i, 0)),
        out_shape=jax.ShapeDtypeStruct((_B, _N1, 64), F32),
        compiler_params=_PAR2,
    )(xw, xw, pif, p1t, *ws)


def _head_kernel(p1_ref, res_ref, cf_ref, cm_ref, q_ref, t_ref,
                 wf1, bf1, wf2, bf2, wm1, bm1, wm2, bm2,
                 wfc, bfc, whq, bhq, wht, bht,
                 ef_ref, em_ref, qo_ref, to_ref):
    p1 = p1_ref[0]
    res = res_ref[0]
    cfe = cf_ref[0]
    cma = cm_ref[0]
    h = jnp.concatenate([p1, res, cfe], axis=1)
    h = _relu(_dot(h, wf1[...]) + bf1[...])
    ef = _relu(_dot(h, wf2[...]) + bf2[...])
    h = jnp.concatenate([cma, ef, p1], axis=1)
    h = _relu(_dot(h, wm1[...]) + bm1[...])
    em = _relu(_dot(h, wm2[...]) + bm2[...])
    ef_ref[0] = ef
    em_ref[0] = em
    mx = jnp.max(em, axis=0, keepdims=True)
    e = jnp.exp(em - mx)
    wcv = e / jnp.sum(e, axis=0, keepdims=True)
    hsum = jnp.sum(ef * wcv, axis=0, keepdims=True)
    hp = _dot(hsum, wfc[...]) + bfc[...]
    qd = _dot(hp, whq[...]) + bhq[...]
    td = _dot(hp, wht[...]) + bht[...]
    qd = qd / (jnp.sqrt(jnp.sum(qd * qd, axis=1, keepdims=True)) + 1e-10)
    qc = q_ref[0]
    a = (qd[:, 0:1], qd[:, 1:2], qd[:, 2:3], qd[:, 3:4])
    b = (qc[:, 0:1], qc[:, 1:2], qc[:, 2:3], qc[:, 3:4])
    qx, qy, qz, qw = _qmul_c(a, b)
    qo_ref[0] = jnp.concatenate([qx, qy, qz, qw], axis=1)
    tc = t_ref[0]
    tx, ty, tz = _warp_cols(tc[:, 0:1], tc[:, 1:2], tc[:, 2:3], qd,
                            td[:, 0:1], td[:, 1:2], td[:, 2:3])
    to_ref[0] = jnp.concatenate([tx, ty, tz], axis=1)


def _head(p1t, res, cfe, cma, q3, t3, p):
    ws = (_flat(p["fp_feat"]) + _flat(p["fp_mask"]) + _flat(p["pose_fc"])
          + _flat(p["head_q"]) + _flat(p["head_t"]))
    return pl.pallas_call(
        _head_kernel,
        grid=(_B,),
        in_specs=[
            pl.BlockSpec((1, _N1, 64), lambda b: (b, 0, 0)),
            pl.BlockSpec((1, _N1, 64), lambda b: (b, 0, 0)),
            pl.BlockSpec((1, _N1, 64), lambda b: (b, 0, 0)),
            pl.BlockSpec((1, _N1, 64), lambda b: (b, 0, 0)),
            pl.BlockSpec((1, 1, 4), lambda b: (b, 0, 0)),
            pl.BlockSpec((1, 1, 3), lambda b: (b, 0, 0)),
        ] + [_wspec1(w.shape) for w in ws],
        out_specs=[
            pl.BlockSpec((1, _N1, 64), lambda b: (b, 0, 0)),
            pl.BlockSpec((1, _N1, 64), lambda b: (b, 0, 0)),
            pl.BlockSpec((1, 1, 4), lambda b: (b, 0, 0)),
            pl.BlockSpec((1, 1, 3), lambda b: (b, 0, 0)),
        ],
        out_shape=[
            jax.ShapeDtypeStruct((_B, _N1, 64), F32),
            jax.ShapeDtypeStruct((_B, _N1, 64), F32),
            jax.ShapeDtypeStruct((_B, 1, 4), F32),
            jax.ShapeDtypeStruct((_B, 1, 3), F32),
        ],
        compiler_params=_PAR1,
    )(p1t, res, cfe, cma, q3, t3, *ws)


def kernel(xyz_f1, points_f1, xyz_f2, points_f2, xyz_f1_prev,
           points_f1_prev, embedding_mask_prev, q_prev, t_prev, params):
    x1t = xyz_f1.transpose(0, 2, 1)
    xpt = xyz_f1_prev.transpose(0, 2, 1)
    p1t = points_f1.transpose(0, 2, 1)
    coarse_feat = _setupconv(x1t, xpt, points_f1_prev.transpose(0, 2, 1),
                             p1t, params["suc_feat_mlp"],
                             params["suc_feat_post"])
    coarse_mask = _setupconv(x1t, xpt,
                             embedding_mask_prev.transpose(0, 2, 1),
                             p1t, params["suc_mask_mlp"],
                             params["suc_mask_post"])
    q3 = q_prev.reshape(_B, 1, 4)
    t3 = t_prev.reshape(_B, 1, 3)
    pi_feat, xw = _cv1(x1t, xyz_f2.transpose(0, 2, 1),
                       points_f2.transpose(0, 2, 1), p1t, q3, t3, params)
    residual = _cv2(xw, pi_feat, p1t, params)
    ef, em, qo, to = _head(p1t, residual, coarse_feat, coarse_mask,
                           q3, t3, params)
    return (qo.reshape(_B, 4), to.reshape(_B, 3),
            ef.transpose(0, 2, 1), em.transpose(0, 2, 1))


# speedup vs baseline: 14.9003x; 1.0553x over previous
import functools

import jax
import jax.numpy as jnp
from jax import lax
from jax.experimental import pallas as pl
from jax.experimental.pallas import tpu as pltpu

_PAR2 = pltpu.CompilerParams(dimension_semantics=("parallel", "parallel"))
_PAR1 = pltpu.CompilerParams(dimension_semantics=("parallel",))

F32 = jnp.float32
_B, _N1, _N2, _N3 = 4, 2048, 2048, 512
_Q = 512


def _dot(a, b):
    return jnp.dot(a, b, preferred_element_type=F32)


def _relu(x):
    return jnp.maximum(x, 0.0)


def _dist(q, r):
    cross = lax.dot_general(q, r, (((1,), (1,)), ((), ())),
                            preferred_element_type=F32)
    return (jnp.sum(q * q, axis=1, keepdims=True) - 2.0 * cross
            + jnp.sum(r * r, axis=1)[None, :])


def _argmin_onehot(d, iota):
    nr = d.shape[1]
    m = jnp.min(d, axis=1, keepdims=True)
    idx = jnp.min(jnp.where(d == m, iota, nr), axis=1, keepdims=True)
    return iota == idx


def _qmul_c(a, b):
    ax, ay, az, aw = a
    bx, by, bz, bw = b
    return (aw * bx + ax * bw + ay * bz - az * by,
            aw * by - ax * bz + ay * bw + az * bx,
            aw * bz + ax * by - ay * bx + az * bw,
            aw * bw - ax * bx - ay * by - az * bz)


def _warp_cols(px, py, pz, q, tx, ty, tz):
    qn = q / (jnp.sqrt(jnp.sum(q * q, axis=1, keepdims=True)) + 1e-10)
    qx, qy, qz, qw = qn[:, 0:1], qn[:, 1:2], qn[:, 2:3], qn[:, 3:4]
    zero = jnp.zeros_like(px)
    r = _qmul_c((qx, qy, qz, qw), (px, py, pz, zero))
    r = _qmul_c(r, (-qx, -qy, -qz, qw))
    return r[0] + tx, r[1] + ty, r[2] + tz


def _wspec(shape):
    return pl.BlockSpec(shape, lambda b, i: tuple(0 for _ in shape))


def _wspec1(shape):
    return pl.BlockSpec(shape, lambda b: tuple(0 for _ in shape))


def _flat(params_list):
    out = []
    for (w, bb) in params_list:
        out.append(w)
        out.append(bb.reshape(1, -1))
    return out


def _suc_kernel(x1_ref, x2_ref, fa_ref, fb_ref, p1_ref,
                w1a, b1a, w2a, b2a, wpa, bpa,
                w1b, b1b, w2b, b2b, wpb, bpb, oa_ref, ob_ref):
    x1 = x1_ref[0]
    x2 = x2_ref[0]
    fa = fa_ref[0]
    fb = fb_ref[0]
    ca = fa.shape[1]
    cb = fb.shape[1]
    fx2 = jnp.concatenate([fa, fb, x2], axis=1)
    d = _dist(x1, x2)
    iota = lax.broadcasted_iota(jnp.int32, d.shape, 1)
    hmax_a = jnp.full((x1.shape[0], w2a.shape[1]), -jnp.inf, F32)
    hmax_b = jnp.full((x1.shape[0], w2b.shape[1]), -jnp.inf, F32)
    for _ in range(8):
        oh = _argmin_onehot(d, iota)
        ohf = jnp.where(oh, 1.0, 0.0).astype(F32)
        g = _dot(ohf, fx2)
        gxyz = g[:, ca + cb:] - x1
        ha = jnp.concatenate([g[:, :ca], gxyz], axis=1)
        ha = _relu(_dot(ha, w1a[...]) + b1a[...])
        ha = _relu(_dot(ha, w2a[...]) + b2a[...])
        hmax_a = jnp.maximum(hmax_a, ha)
        hb = jnp.concatenate([g[:, ca:ca + cb], gxyz], axis=1)
        hb = _relu(_dot(hb, w1b[...]) + b1b[...])
        hb = _relu(_dot(hb, w2b[...]) + b2b[...])
        hmax_b = jnp.maximum(hmax_b, hb)
        d = jnp.where(oh, jnp.inf, d)
    p1 = p1_ref[0]
    hpa = jnp.concatenate([hmax_a, p1], axis=1)
    oa_ref[0] = _relu(_dot(hpa, wpa[...]) + bpa[...])
    hpb = jnp.concatenate([hmax_b, p1], axis=1)
    ob_ref[0] = _relu(_dot(hpb, wpb[...]) + bpb[...])


def _setupconv(x1t, x2t, fat, fbt, p1t, p):
    ca = fat.shape[-1]
    cb = fbt.shape[-1]
    nblk = _N1 // _Q
    ws = (_flat(p["suc_feat_mlp"]) + _flat(p["suc_feat_post"])
          + _flat(p["suc_mask_mlp"]) + _flat(p["suc_mask_post"]))
    return pl.pallas_call(
        _suc_kernel,
        grid=(_B, nblk),
        in_specs=[
            pl.BlockSpec((1, _Q, 3), lambda b, i: (b, i, 0)),
            pl.BlockSpec((1, _N3, 3), lambda b, i: (b, 0, 0)),
            pl.BlockSpec((1, _N3, ca), lambda b, i: (b, 0, 0)),
            pl.BlockSpec((1, _N3, cb), lambda b, i: (b, 0, 0)),
            pl.BlockSpec((1, _Q, 64), lambda b, i: (b, i, 0)),
        ] + [_wspec(w.shape) for w in ws],
        out_specs=[
            pl.BlockSpec((1, _Q, 64), lambda b, i: (b, i, 0)),
            pl.BlockSpec((1, _Q, 64), lambda b, i: (b, i, 0)),
        ],
        out_shape=[
            jax.ShapeDtypeStruct((_B, _N1, 64), F32),
            jax.ShapeDtypeStruct((_B, _N1, 64), F32),
        ],
        compiler_params=_PAR2,
    )(x1t, x2t, fat, fbt, p1t, *ws)


def _cv1_kernel(x1_ref, x2_ref, f2_ref, p1_ref, q_ref, t_ref,
                wm1, bm1, wm2, bm2, wm3, bm3, we, be,
                wq1, bq1, wq2, bq2, pif_ref, xw_ref):
    x1 = x1_ref[0]
    q = q_ref[0]
    t = t_ref[0]
    wx, wy, wz = _warp_cols(x1[:, 0:1], x1[:, 1:2], x1[:, 2:3], q,
                            t[:, 0:1], t[:, 1:2], t[:, 2:3])
    xw = jnp.concatenate([wx, wy, wz], axis=1)
    x2 = x2_ref[0]
    f2 = f2_ref[0]
    p1 = p1_ref[0]
    xf2 = jnp.concatenate([x2, f2], axis=1)
    d = _dist(xw, x2)
    iota = lax.broadcasted_iota(jnp.int32, d.shape, 1)
    feats = []
    wqs = []
    for _ in range(6):
        oh = _argmin_onehot(d, iota)
        ohf = jnp.where(oh, 1.0, 0.0).astype(F32)
        g = _dot(ohf, xf2)
        qi_xyz = g[:, 0:3]
        qi_f = g[:, 3:]
        diff = qi_xyz - xw
        euc = jnp.sqrt(jnp.sum(diff * diff, axis=1, keepdims=True) + 1e-10)
        xyz_cat = jnp.concatenate([xw, qi_xyz, diff, euc], axis=1)
        h = jnp.concatenate([xyz_cat, p1, qi_f], axis=1)
        h = _relu(_dot(h, wm1[...]) + bm1[...])
        h = _relu(_dot(h, wm2[...]) + bm2[...])
        feat = _relu(_dot(h, wm3[...]) + bm3[...])
        enc = _relu(_dot(xyz_cat, we[...]) + be[...])
        hq = jnp.concatenate([enc, feat], axis=1)
        hq = _relu(_dot(hq, wq1[...]) + bq1[...])
        wq = _relu(_dot(hq, wq2[...]) + bq2[...])
        feats.append(feat)
        wqs.append(wq)
        d = jnp.where(oh, jnp.inf, d)
    wmax = functools.reduce(jnp.maximum, wqs)
    es = [jnp.exp(w - wmax) for w in wqs]
    ssum = functools.reduce(jnp.add, es)
    pif = functools.reduce(jnp.add, [e * f for e, f in zip(es, feats)]) / ssum
    pif_ref[0] = pif
    xw_ref[0] = xw


def _cv1(x1t, x2t, f2t, p1t, q3, t3, p):
    nblk = _N1 // _Q
    ws = _flat(p["cv_mlp1"]) + _flat(p["cv_enc1"]) + _flat(p["cv_w_q"])
    return pl.pallas_call(
        _cv1_kernel,
        grid=(_B, nblk),
        in_specs=[
            pl.BlockSpec((1, _Q, 3), lambda b, i: (b, i, 0)),
            pl.BlockSpec((1, _N2, 3), lambda b, i: (b, 0, 0)),
            pl.BlockSpec((1, _N2, 64), lambda b, i: (b, 0, 0)),
            pl.BlockSpec((1, _Q, 64), lambda b, i: (b, i, 0)),
            pl.BlockSpec((1, 1, 4), lambda b, i: (b, 0, 0)),
            pl.BlockSpec((1, 1, 3), lambda b, i: (b, 0, 0)),
        ] + [_wspec(w.shape) for w in ws],
        out_specs=[
            pl.BlockSpec((1, _Q, 64), lambda b, i: (b, i, 0)),
            pl.BlockSpec((1, _Q, 3), lambda b, i: (b, i, 0)),
        ],
        out_shape=[
            jax.ShapeDtypeStruct((_B, _N1, 64), F32),
            jax.ShapeDtypeStruct((_B, _N1, 3), F32),
        ],
        compiler_params=_PAR2,
    )(x1t, x2t, f2t, p1t, q3, t3, *ws)


def _cv2_kernel(xw_ref, xr_ref, fr_ref, p1_ref,
                we, be, wp1, bp1, wp2, bp2, o_ref):
    xw = xw_ref[0]
    xr = xr_ref[0]
    fr = fr_ref[0]
    p1 = p1_ref[0]
    xfr = jnp.concatenate([xr, fr], axis=1)
    d = _dist(xw, xr)
    iota = lax.broadcasted_iota(jnp.int32, d.shape, 1)
    gs = []
    wps = []
    for _ in range(4):
        oh = _argmin_onehot(d, iota)
        ohf = jnp.where(oh, 1.0, 0.0).astype(F32)
        g = _dot(ohf, xfr)
        pc_xyz = g[:, 0:3]
        pc_g = g[:, 3:]
        d2 = pc_xyz - xw
        e2 = jnp.sqrt(jnp.sum(d2 * d2, axis=1, keepdims=True) + 1e-10)
        xyz_cat2 = jnp.concatenate([xw, pc_xyz, d2, e2], axis=1)
        enc2 = _relu(_dot(xyz_cat2, we[...]) + be[...])
        h = jnp.concatenate([enc2, p1, pc_g], axis=1)
        h = _relu(_dot(h, wp1[...]) + bp1[...])
        wpv = _relu(_dot(h, wp2[...]) + bp2[...])
        gs.append(pc_g)
        wps.append(wpv)
        d = jnp.where(oh, jnp.inf, d)
    wmax = functools.reduce(jnp.maximum, wps)
    es = [jnp.exp(w - wmax) for w in wps]
    ssum = functools.reduce(jnp.add, es)
    o_ref[0] = functools.reduce(
        jnp.add, [e * g for e, g in zip(es, gs)]) / ssum


def _cv2(xw, pif, p1t, p):
    nblk = _N1 // _Q
    ws = _flat(p["cv_enc2"]) + _flat(p["cv_w_p"])
    return pl.pallas_call(
        _cv2_kernel,
        grid=(_B, nblk),
        in_specs=[
            pl.BlockSpec((1, _Q, 3), lambda b, i: (b, i, 0)),
            pl.BlockSpec((1, _N1, 3), lambda b, i: (b, 0, 0)),
            pl.BlockSpec((1, _N1, 64), lambda b, i: (b, 0, 0)),
            pl.BlockSpec((1, _Q, 64), lambda b, i: (b, i, 0)),
        ] + [_wspec(w.shape) for w in ws],
        out_specs=pl.BlockSpec((1, _Q, 64), lambda b, i: (b, i, 0)),
        out_shape=jax.ShapeDtypeStruct((_B, _N1, 64), F32),
        compiler_params=_PAR2,
    )(xw, xw, pif, p1t, *ws)


def _head_kernel(p1_ref, res_ref, cf_ref, cm_ref, q_ref, t_ref,
                 wf1, bf1, wf2, bf2, wm1, bm1, wm2, bm2,
                 wfc, bfc, whq, bhq, wht, bht,
                 ef_ref, em_ref, qo_ref, to_ref):
    p1 = p1_ref[0]
    res = res_ref[0]
    cfe = cf_ref[0]
    cma = cm_ref[0]
    h = jnp.concatenate([p1, res, cfe], axis=1)
    h = _relu(_dot(h, wf1[...]) + bf1[...])
    ef = _relu(_dot(h, wf2[...]) + bf2[...])
    h = jnp.concatenate([cma, ef, p1], axis=1)
    h = _relu(_dot(h, wm1[...]) + bm1[...])
    em = _relu(_dot(h, wm2[...]) + bm2[...])
    ef_ref[0] = ef
    em_ref[0] = em
    mx = jnp.max(em, axis=0, keepdims=True)
    e = jnp.exp(em - mx)
    wcv = e / jnp.sum(e, axis=0, keepdims=True)
    hsum = jnp.sum(ef * wcv, axis=0, keepdims=True)
    hp = _dot(hsum, wfc[...]) + bfc[...]
    qd = _dot(hp, whq[...]) + bhq[...]
    td = _dot(hp, wht[...]) + bht[...]
    qd = qd / (jnp.sqrt(jnp.sum(qd * qd, axis=1, keepdims=True)) + 1e-10)
    qc = q_ref[0]
    a = (qd[:, 0:1], qd[:, 1:2], qd[:, 2:3], qd[:, 3:4])
    b = (qc[:, 0:1], qc[:, 1:2], qc[:, 2:3], qc[:, 3:4])
    qx, qy, qz, qw = _qmul_c(a, b)
    qo_ref[0] = jnp.concatenate([qx, qy, qz, qw], axis=1)
    tc = t_ref[0]
    tx, ty, tz = _warp_cols(tc[:, 0:1], tc[:, 1:2], tc[:, 2:3], qd,
                            td[:, 0:1], td[:, 1:2], td[:, 2:3])
    to_ref[0] = jnp.concatenate([tx, ty, tz], axis=1)


def _head(p1t, res, cfe, cma, q3, t3, p):
    ws = (_flat(p["fp_feat"]) + _flat(p["fp_mask"]) + _flat(p["pose_fc"])
          + _flat(p["head_q"]) + _flat(p["head_t"]))
    return pl.pallas_call(
        _head_kernel,
        grid=(_B,),
        in_specs=[
            pl.BlockSpec((1, _N1, 64), lambda b: (b, 0, 0)),
            pl.BlockSpec((1, _N1, 64), lambda b: (b, 0, 0)),
            pl.BlockSpec((1, _N1, 64), lambda b: (b, 0, 0)),
            pl.BlockSpec((1, _N1, 64), lambda b: (b, 0, 0)),
            pl.BlockSpec((1, 1, 4), lambda b: (b, 0, 0)),
            pl.BlockSpec((1, 1, 3), lambda b: (b, 0, 0)),
        ] + [_wspec1(w.shape) for w in ws],
        out_specs=[
            pl.BlockSpec((1, _N1, 64), lambda b: (b, 0, 0)),
            pl.BlockSpec((1, _N1, 64), lambda b: (b, 0, 0)),
            pl.BlockSpec((1, 1, 4), lambda b: (b, 0, 0)),
            pl.BlockSpec((1, 1, 3), lambda b: (b, 0, 0)),
        ],
        out_shape=[
            jax.ShapeDtypeStruct((_B, _N1, 64), F32),
            jax.ShapeDtypeStruct((_B, _N1, 64), F32),
            jax.ShapeDtypeStruct((_B, 1, 4), F32),
            jax.ShapeDtypeStruct((_B, 1, 3), F32),
        ],
        compiler_params=_PAR1,
    )(p1t, res, cfe, cma, q3, t3, *ws)


def kernel(xyz_f1, points_f1, xyz_f2, points_f2, xyz_f1_prev,
           points_f1_prev, embedding_mask_prev, q_prev, t_prev, params):
    x1t = xyz_f1.transpose(0, 2, 1)
    xpt = xyz_f1_prev.transpose(0, 2, 1)
    p1t = points_f1.transpose(0, 2, 1)
    coarse_feat, coarse_mask = _setupconv(
        x1t, xpt, points_f1_prev.transpose(0, 2, 1),
        embedding_mask_prev.transpose(0, 2, 1), p1t, params)
    q3 = q_prev.reshape(_B, 1, 4)
    t3 = t_prev.reshape(_B, 1, 3)
    pi_feat, xw = _cv1(x1t, xyz_f2.transpose(0, 2, 1),
                       points_f2.transpose(0, 2, 1), p1t, q3, t3, params)
    residual = _cv2(xw, pi_feat, p1t, params)
    ef, em, qo, to = _head(p1t, residual, coarse_feat, coarse_mask,
                           q3, t3, params)
    return (qo.reshape(_B, 4), to.reshape(_B, 3),
            ef.transpose(0, 2, 1), em.transpose(0, 2, 1))


# speedup vs baseline: 15.5823x; 1.0458x over previous
import functools

import jax
import jax.numpy as jnp
from jax import lax
from jax.experimental import pallas as pl
from jax.experimental.pallas import tpu as pltpu

_PAR2 = pltpu.CompilerParams(dimension_semantics=("parallel", "parallel"))
_PAR1 = pltpu.CompilerParams(dimension_semantics=("parallel",))

F32 = jnp.float32
_B, _N1, _N2, _N3 = 4, 2048, 2048, 512
_Q = 512


def _dot(a, b):
    return jnp.dot(a, b, preferred_element_type=F32)


def _relu(x):
    return jnp.maximum(x, 0.0)


def _dist(q, r):
    cross = lax.dot_general(q, r, (((1,), (1,)), ((), ())),
                            preferred_element_type=F32)
    return (jnp.sum(q * q, axis=1, keepdims=True) - 2.0 * cross
            + jnp.sum(r * r, axis=1)[None, :])


def _argmin_onehot(d, iota):
    nr = d.shape[1]
    m = jnp.min(d, axis=1, keepdims=True)
    idx = jnp.min(jnp.where(d == m, iota, nr), axis=1, keepdims=True)
    return iota == idx


def _argmin_onehot_wide(d, iota):
    idx = jnp.argmin(d, axis=1).reshape(-1, 1)
    return iota == idx


def _qmul_c(a, b):
    ax, ay, az, aw = a
    bx, by, bz, bw = b
    return (aw * bx + ax * bw + ay * bz - az * by,
            aw * by - ax * bz + ay * bw + az * bx,
            aw * bz + ax * by - ay * bx + az * bw,
            aw * bw - ax * bx - ay * by - az * bz)


def _warp_cols(px, py, pz, q, tx, ty, tz):
    qn = q / (jnp.sqrt(jnp.sum(q * q, axis=1, keepdims=True)) + 1e-10)
    qx, qy, qz, qw = qn[:, 0:1], qn[:, 1:2], qn[:, 2:3], qn[:, 3:4]
    zero = jnp.zeros_like(px)
    r = _qmul_c((qx, qy, qz, qw), (px, py, pz, zero))
    r = _qmul_c(r, (-qx, -qy, -qz, qw))
    return r[0] + tx, r[1] + ty, r[2] + tz


def _wspec(shape):
    return pl.BlockSpec(shape, lambda b, i: tuple(0 for _ in shape))


def _wspec1(shape):
    return pl.BlockSpec(shape, lambda b: tuple(0 for _ in shape))


def _flat(params_list):
    out = []
    for (w, bb) in params_list:
        out.append(w)
        out.append(bb.reshape(1, -1))
    return out


def _suc_kernel(x1_ref, x2_ref, fa_ref, fb_ref, p1_ref,
                w1a, b1a, w2a, b2a, wpa, bpa,
                w1b, b1b, w2b, b2b, wpb, bpb, oa_ref, ob_ref):
    x1 = x1_ref[0]
    x2 = x2_ref[0]
    fa = fa_ref[0]
    fb = fb_ref[0]
    ca = fa.shape[1]
    cb = fb.shape[1]
    fx2 = jnp.concatenate([fa, fb, x2], axis=1)
    d = _dist(x1, x2)
    iota = lax.broadcasted_iota(jnp.int32, d.shape, 1)
    hmax_a = jnp.full((x1.shape[0], w2a.shape[1]), -jnp.inf, F32)
    hmax_b = jnp.full((x1.shape[0], w2b.shape[1]), -jnp.inf, F32)
    for _ in range(8):
        oh = _argmin_onehot(d, iota)
        ohf = jnp.where(oh, 1.0, 0.0).astype(F32)
        g = _dot(ohf, fx2)
        gxyz = g[:, ca + cb:] - x1
        ha = jnp.concatenate([g[:, :ca], gxyz], axis=1)
        ha = _relu(_dot(ha, w1a[...]) + b1a[...])
        ha = _relu(_dot(ha, w2a[...]) + b2a[...])
        hmax_a = jnp.maximum(hmax_a, ha)
        hb = jnp.concatenate([g[:, ca:ca + cb], gxyz], axis=1)
        hb = _relu(_dot(hb, w1b[...]) + b1b[...])
        hb = _relu(_dot(hb, w2b[...]) + b2b[...])
        hmax_b = jnp.maximum(hmax_b, hb)
        d = jnp.where(oh, jnp.inf, d)
    p1 = p1_ref[0]
    hpa = jnp.concatenate([hmax_a, p1], axis=1)
    oa_ref[0] = _relu(_dot(hpa, wpa[...]) + bpa[...])
    hpb = jnp.concatenate([hmax_b, p1], axis=1)
    ob_ref[0] = _relu(_dot(hpb, wpb[...]) + bpb[...])


def _setupconv(x1t, x2t, fat, fbt, p1t, p):
    ca = fat.shape[-1]
    cb = fbt.shape[-1]
    nblk = _N1 // _Q
    ws = (_flat(p["suc_feat_mlp"]) + _flat(p["suc_feat_post"])
          + _flat(p["suc_mask_mlp"]) + _flat(p["suc_mask_post"]))
    return pl.pallas_call(
        _suc_kernel,
        grid=(_B, nblk),
        in_specs=[
            pl.BlockSpec((1, _Q, 3), lambda b, i: (b, i, 0)),
            pl.BlockSpec((1, _N3, 3), lambda b, i: (b, 0, 0)),
            pl.BlockSpec((1, _N3, ca), lambda b, i: (b, 0, 0)),
            pl.BlockSpec((1, _N3, cb), lambda b, i: (b, 0, 0)),
            pl.BlockSpec((1, _Q, 64), lambda b, i: (b, i, 0)),
        ] + [_wspec(w.shape) for w in ws],
        out_specs=[
            pl.BlockSpec((1, _Q, 64), lambda b, i: (b, i, 0)),
            pl.BlockSpec((1, _Q, 64), lambda b, i: (b, i, 0)),
        ],
        out_shape=[
            jax.ShapeDtypeStruct((_B, _N1, 64), F32),
            jax.ShapeDtypeStruct((_B, _N1, 64), F32),
        ],
        compiler_params=_PAR2,
    )(x1t, x2t, fat, fbt, p1t, *ws)


def _cv1_kernel(x1_ref, x2_ref, f2_ref, p1_ref, q_ref, t_ref,
                wm1, bm1, wm2, bm2, wm3, bm3, we, be,
                wq1, bq1, wq2, bq2, pif_ref, xw_ref):
    x1 = x1_ref[0]
    q = q_ref[0]
    t = t_ref[0]
    wx, wy, wz = _warp_cols(x1[:, 0:1], x1[:, 1:2], x1[:, 2:3], q,
                            t[:, 0:1], t[:, 1:2], t[:, 2:3])
    xw = jnp.concatenate([wx, wy, wz], axis=1)
    x2 = x2_ref[0]
    f2 = f2_ref[0]
    p1 = p1_ref[0]
    xf2 = jnp.concatenate([x2, f2], axis=1)
    d = _dist(xw, x2)
    iota = lax.broadcasted_iota(jnp.int32, d.shape, 1)
    feats = []
    wqs = []
    for _ in range(6):
        oh = _argmin_onehot_wide(d, iota)
        ohf = jnp.where(oh, 1.0, 0.0).astype(F32)
        g = _dot(ohf, xf2)
        qi_xyz = g[:, 0:3]
        qi_f = g[:, 3:]
        diff = qi_xyz - xw
        euc = jnp.sqrt(jnp.sum(diff * diff, axis=1, keepdims=True) + 1e-10)
        xyz_cat = jnp.concatenate([xw, qi_xyz, diff, euc], axis=1)
        h = jnp.concatenate([xyz_cat, p1, qi_f], axis=1)
        h = _relu(_dot(h, wm1[...]) + bm1[...])
        h = _relu(_dot(h, wm2[...]) + bm2[...])
        feat = _relu(_dot(h, wm3[...]) + bm3[...])
        enc = _relu(_dot(xyz_cat, we[...]) + be[...])
        hq = jnp.concatenate([enc, feat], axis=1)
        hq = _relu(_dot(hq, wq1[...]) + bq1[...])
        wq = _relu(_dot(hq, wq2[...]) + bq2[...])
        feats.append(feat)
        wqs.append(wq)
        d = jnp.where(oh, jnp.inf, d)
    wmax = functools.reduce(jnp.maximum, wqs)
    es = [jnp.exp(w - wmax) for w in wqs]
    ssum = functools.reduce(jnp.add, es)
    pif = functools.reduce(jnp.add, [e * f for e, f in zip(es, feats)]) / ssum
    pif_ref[0] = pif
    xw_ref[0] = xw


def _cv1(x1t, x2t, f2t, p1t, q3, t3, p):
    nblk = _N1 // _Q
    ws = _flat(p["cv_mlp1"]) + _flat(p["cv_enc1"]) + _flat(p["cv_w_q"])
    return pl.pallas_call(
        _cv1_kernel,
        grid=(_B, nblk),
        in_specs=[
            pl.BlockSpec((1, _Q, 3), lambda b, i: (b, i, 0)),
            pl.BlockSpec((1, _N2, 3), lambda b, i: (b, 0, 0)),
            pl.BlockSpec((1, _N2, 64), lambda b, i: (b, 0, 0)),
            pl.BlockSpec((1, _Q, 64), lambda b, i: (b, i, 0)),
            pl.BlockSpec((1, 1, 4), lambda b, i: (b, 0, 0)),
            pl.BlockSpec((1, 1, 3), lambda b, i: (b, 0, 0)),
        ] + [_wspec(w.shape) for w in ws],
        out_specs=[
            pl.BlockSpec((1, _Q, 64), lambda b, i: (b, i, 0)),
            pl.BlockSpec((1, _Q, 3), lambda b, i: (b, i, 0)),
        ],
        out_shape=[
            jax.ShapeDtypeStruct((_B, _N1, 64), F32),
            jax.ShapeDtypeStruct((_B, _N1, 3), F32),
        ],
        compiler_params=_PAR2,
    )(x1t, x2t, f2t, p1t, q3, t3, *ws)


def _cv2_kernel(xw_ref, xr_ref, fr_ref, p1_ref,
                we, be, wp1, bp1, wp2, bp2, o_ref):
    xw = xw_ref[0]
    xr = xr_ref[0]
    fr = fr_ref[0]
    p1 = p1_ref[0]
    xfr = jnp.concatenate([xr, fr], axis=1)
    d = _dist(xw, xr)
    iota = lax.broadcasted_iota(jnp.int32, d.shape, 1)
    gs = []
    wps = []
    for _ in range(4):
        oh = _argmin_onehot_wide(d, iota)
        ohf = jnp.where(oh, 1.0, 0.0).astype(F32)
        g = _dot(ohf, xfr)
        pc_xyz = g[:, 0:3]
        pc_g = g[:, 3:]
        d2 = pc_xyz - xw
        e2 = jnp.sqrt(jnp.sum(d2 * d2, axis=1, keepdims=True) + 1e-10)
        xyz_cat2 = jnp.concatenate([xw, pc_xyz, d2, e2], axis=1)
        enc2 = _relu(_dot(xyz_cat2, we[...]) + be[...])
        h = jnp.concatenate([enc2, p1, pc_g], axis=1)
        h = _relu(_dot(h, wp1[...]) + bp1[...])
        wpv = _relu(_dot(h, wp2[...]) + bp2[...])
        gs.append(pc_g)
        wps.append(wpv)
        d = jnp.where(oh, jnp.inf, d)
    wmax = functools.reduce(jnp.maximum, wps)
    es = [jnp.exp(w - wmax) for w in wps]
    ssum = functools.reduce(jnp.add, es)
    o_ref[0] = functools.reduce(
        jnp.add, [e * g for e, g in zip(es, gs)]) / ssum


def _cv2(xw, pif, p1t, p):
    nblk = _N1 // _Q
    ws = _flat(p["cv_enc2"]) + _flat(p["cv_w_p"])
    return pl.pallas_call(
        _cv2_kernel,
        grid=(_B, nblk),
        in_specs=[
            pl.BlockSpec((1, _Q, 3), lambda b, i: (b, i, 0)),
            pl.BlockSpec((1, _N1, 3), lambda b, i: (b, 0, 0)),
            pl.BlockSpec((1, _N1, 64), lambda b, i: (b, 0, 0)),
            pl.BlockSpec((1, _Q, 64), lambda b, i: (b, i, 0)),
        ] + [_wspec(w.shape) for w in ws],
        out_specs=pl.BlockSpec((1, _Q, 64), lambda b, i: (b, i, 0)),
        out_shape=jax.ShapeDtypeStruct((_B, _N1, 64), F32),
        compiler_params=_PAR2,
    )(xw, xw, pif, p1t, *ws)


def _head_kernel(p1_ref, res_ref, cf_ref, cm_ref, q_ref, t_ref,
                 wf1, bf1, wf2, bf2, wm1, bm1, wm2, bm2,
                 wfc, bfc, whq, bhq, wht, bht,
                 ef_ref, em_ref, qo_ref, to_ref):
    p1 = p1_ref[0]
    res = res_ref[0]
    cfe = cf_ref[0]
    cma = cm_ref[0]
    h = jnp.concatenate([p1, res, cfe], axis=1)
    h = _relu(_dot(h, wf1[...]) + bf1[...])
    ef = _relu(_dot(h, wf2[...]) + bf2[...])
    h = jnp.concatenate([cma, ef, p1], axis=1)
    h = _relu(_dot(h, wm1[...]) + bm1[...])
    em = _relu(_dot(h, wm2[...]) + bm2[...])
    ef_ref[0] = ef
    em_ref[0] = em
    mx = jnp.max(em, axis=0, keepdims=True)
    e = jnp.exp(em - mx)
    wcv = e / jnp.sum(e, axis=0, keepdims=True)
    hsum = jnp.sum(ef * wcv, axis=0, keepdims=True)
    hp = _dot(hsum, wfc[...]) + bfc[...]
    qd = _dot(hp, whq[...]) + bhq[...]
    td = _dot(hp, wht[...]) + bht[...]
    qd = qd / (jnp.sqrt(jnp.sum(qd * qd, axis=1, keepdims=True)) + 1e-10)
    qc = q_ref[0]
    a = (qd[:, 0:1], qd[:, 1:2], qd[:, 2:3], qd[:, 3:4])
    b = (qc[:, 0:1], qc[:, 1:2], qc[:, 2:3], qc[:, 3:4])
    qx, qy, qz, qw = _qmul_c(a, b)
    qo_ref[0] = jnp.concatenate([qx, qy, qz, qw], axis=1)
    tc = t_ref[0]
    tx, ty, tz = _warp_cols(tc[:, 0:1], tc[:, 1:2], tc[:, 2:3], qd,
                            td[:, 0:1], td[:, 1:2], td[:, 2:3])
    to_ref[0] = jnp.concatenate([tx, ty, tz], axis=1)


def _head(p1t, res, cfe, cma, q3, t3, p):
    ws = (_flat(p["fp_feat"]) + _flat(p["fp_mask"]) + _flat(p["pose_fc"])
          + _flat(p["head_q"]) + _flat(p["head_t"]))
    return pl.pallas_call(
        _head_kernel,
        grid=(_B,),
        in_specs=[
            pl.BlockSpec((1, _N1, 64), lambda b: (b, 0, 0)),
            pl.BlockSpec((1, _N1, 64), lambda b: (b, 0, 0)),
            pl.BlockSpec((1, _N1, 64), lambda b: (b, 0, 0)),
            pl.BlockSpec((1, _N1, 64), lambda b: (b, 0, 0)),
            pl.BlockSpec((1, 1, 4), lambda b: (b, 0, 0)),
            pl.BlockSpec((1, 1, 3), lambda b: (b, 0, 0)),
        ] + [_wspec1(w.shape) for w in ws],
        out_specs=[
            pl.BlockSpec((1, _N1, 64), lambda b: (b, 0, 0)),
            pl.BlockSpec((1, _N1, 64), lambda b: (b, 0, 0)),
            pl.BlockSpec((1, 1, 4), lambda b: (b, 0, 0)),
            pl.BlockSpec((1, 1, 3), lambda b: (b, 0, 0)),
        ],
        out_shape=[
            jax.ShapeDtypeStruct((_B, _N1, 64), F32),
            jax.ShapeDtypeStruct((_B, _N1, 64), F32),
            jax.ShapeDtypeStruct((_B, 1, 4), F32),
            jax.ShapeDtypeStruct((_B, 1, 3), F32),
        ],
        compiler_params=_PAR1,
    )(p1t, res, cfe, cma, q3, t3, *ws)


def kernel(xyz_f1, points_f1, xyz_f2, points_f2, xyz_f1_prev,
           points_f1_prev, embedding_mask_prev, q_prev, t_prev, params):
    x1t = xyz_f1.transpose(0, 2, 1)
    xpt = xyz_f1_prev.transpose(0, 2, 1)
    p1t = points_f1.transpose(0, 2, 1)
    coarse_feat, coarse_mask = _setupconv(
        x1t, xpt, points_f1_prev.transpose(0, 2, 1),
        embedding_mask_prev.transpose(0, 2, 1), p1t, params)
    q3 = q_prev.reshape(_B, 1, 4)
    t3 = t_prev.reshape(_B, 1, 3)
    pi_feat, xw = _cv1(x1t, xyz_f2.transpose(0, 2, 1),
                       points_f2.transpose(0, 2, 1), p1t, q3, t3, params)
    residual = _cv2(xw, pi_feat, p1t, params)
    ef, em, qo, to = _head(p1t, residual, coarse_feat, coarse_mask,
                           q3, t3, params)
    return (qo.reshape(_B, 4), to.reshape(_B, 3),
            ef.transpose(0, 2, 1), em.transpose(0, 2, 1))
